# baseline (speedup 1.0000x reference)
import functools

import jax
import jax.numpy as jnp
from jax import lax
from jax.experimental import pallas as pl
from jax.experimental.pallas import tpu as pltpu
from jax.experimental.pallas import tpu_sc as plsc

B, N, C = 8, 65536, 64
M, K = 32768, 27

NC, NS, L = 2, 16, 16
NW = NC * NS
M_PER_W = M // NW
CR = 256
HALVES = M_PER_W // CR
G = CR // 128
MB = CR // 128
CH = C // 8


def _sparse_select(features_flat, batches, offsets_t):
    mesh = plsc.VectorSubcoreMesh(core_axis_name="c", subcore_axis_name="s")

    @functools.partial(
        pl.kernel,
        mesh=mesh,
        compiler_params=pltpu.CompilerParams(use_tc_tiling_on_sc=False,
                                             needs_layout_passes=False),
        out_type=jax.ShapeDtypeStruct((K * CH, M // 128, 8 * 128), jnp.float32),
        scratch_types=[
            pltpu.VMEM((M_PER_W,), jnp.int32),
            pltpu.VMEM((K, M_PER_W), jnp.int32),
            pltpu.VMEM((G, 128), jnp.int32),
            pltpu.VMEM((G, 128), jnp.int32),
            pltpu.VMEM((MB, 128, C), jnp.float32),
            pltpu.VMEM((MB, 128, C), jnp.float32),
            pltpu.VMEM((CH, MB, 8 * 128), jnp.float32),
            pltpu.VMEM((CH, MB, 8 * 128), jnp.float32),
            pltpu.SemaphoreType.DMA,
            pltpu.SemaphoreType.DMA,
            pltpu.SemaphoreType.DMA,
            pltpu.SemaphoreType.DMA,
        ],
    )
    def body(feat_hbm, batches_hbm, offs_hbm, out_hbm,
             bat_v, offs_v, idx0, idx1, rows0, rows1, rx0, rx1,
             sem_g0, sem_g1, sem_w0, sem_w1):
        wid = lax.axis_index("s") * NC + lax.axis_index("c")
        mw0 = wid * M_PER_W
        pltpu.sync_copy(batches_hbm.at[pl.ds(mw0, M_PER_W)], bat_v)
        pltpu.sync_copy(offs_hbm.at[:, pl.ds(mw0, M_PER_W)], offs_v)
        iota16 = lax.iota(jnp.int32, 16)
        lane_hi = iota16 >> 3
        inner_base = (iota16 & 7) * 128

        def compute_idx(k, h, idx_v):
            for j in range(CR // 16):
                m_loc = h * CR + j * 16
                b = bat_v[pl.ds(m_loc, 16)]
                off = offs_v[k, pl.ds(m_loc, 16)]
                idx_v[j >> 3, pl.ds((j & 7) * 16, 16)] = b * N + off

        def fire_gathers(idx_v, rows_v, sem):
            return [
                pltpu.async_copy(feat_hbm.at[idx_v.at[g]], rows_v.at[g], sem)
                for g in range(G)
            ]

        def transpose(rows_v, rx_v):
            for mh in range(MB):
                mh_vec = jnp.full((16,), mh, jnp.int32)

                def tbody(ml, carry):
                    inner = inner_base + ml
                    for j in range(4):
                        data = rows_v[mh, ml, pl.ds(j * 16, 16)]
                        ch = 2 * j + lane_hi
                        plsc.store_scatter(rx_v, [ch, mh_vec, inner], data)
                    return carry

                lax.fori_loop(0, 128, tbody, 0, unroll=2)

        def fire_wb(k, h, rx_v, sem):
            mb0 = wid * (M_PER_W // 128) + h * MB
            return [
                pltpu.async_copy(
                    rx_v.at[ch],
                    out_hbm.at[k * CH + ch, pl.ds(mb0, MB), :],
                    sem,
                )
                for ch in range(CH)
            ]

        def wb_wait(rx_v, sem):
            for ch in range(CH):
                pltpu.make_async_copy(
                    rx_v.at[ch], out_hbm.at[0, pl.ds(0, MB), :], sem).wait()

        def per_k(k, carry):
            compute_idx(k, 0, idx0)
            ga = fire_gathers(idx0, rows0, sem_g0)
            compute_idx(k, 1, idx1)
            gb = fire_gathers(idx1, rows1, sem_g1)

            for h in ga:
                h.wait()

            @pl.when(k > 0)
            def _():
                wb_wait(rx0, sem_w0)
            transpose(rows0, rx0)
            fire_wb(k, 0, rx0, sem_w0)
            compute_idx(k, 2, idx0)
            ga2 = fire_gathers(idx0, rows0, sem_g0)

            for h in gb:
                h.wait()

            @pl.when(k > 0)
            def _():
                wb_wait(rx1, sem_w1)
            transpose(rows1, rx1)
            fire_wb(k, 1, rx1, sem_w1)
            compute_idx(k, 3, idx1)
            gb2 = fire_gathers(idx1, rows1, sem_g1)

            for h in ga2:
                h.wait()
            wb_wait(rx0, sem_w0)
            transpose(rows0, rx0)
            fire_wb(k, 2, rx0, sem_w0)

            for h in gb2:
                h.wait()
            wb_wait(rx1, sem_w1)
            transpose(rows1, rx1)
            fire_wb(k, 3, rx1, sem_w1)
            return carry

        lax.fori_loop(0, K, per_k, 0)
        wb_wait(rx0, sem_w0)
        wb_wait(rx1, sem_w1)

    return body(features_flat, batches, offsets_t)


def kernel(features, batches, offsets):
    features_flat = features.reshape(B * N, C)
    offsets_t = offsets.astype(jnp.int32).T
    raw = _sparse_select(features_flat, batches.astype(jnp.int32), offsets_t)
    out5d = raw.reshape(K, CH, M // 128, 8, 128)
    return out5d.transpose(2, 4, 0, 1, 3).reshape(M, K, C)
